# Initial kernel scaffold; baseline (speedup 1.0000x reference)
import jax
import jax.numpy as jnp
from jax.experimental import pallas as pl


def kernel(indices, embeddings):
    raise NotImplementedError("write your pallas kernel here")


# speedup vs baseline: 1.7951x; 1.7951x over previous
import functools

import jax
import jax.numpy as jnp
from jax import lax
from jax.experimental import pallas as pl
from jax.experimental.pallas import tpu as pltpu
from jax.experimental.pallas import tpu_sc as plsc


def _make_gather(n_total: int, vocab: int, dim: int, num_workers: int,
                 num_cores: int, chunk: int):
  n_per_w = n_total // num_workers
  n_chunks = n_per_w // chunk
  mesh = plsc.VectorSubcoreMesh(core_axis_name="c", subcore_axis_name="s")

  @functools.partial(
      pl.kernel,
      out_type=jax.ShapeDtypeStruct((n_total, dim), jnp.float32),
      mesh=mesh,
      scratch_types=[
          pltpu.VMEM((chunk,), jnp.int32),
          pltpu.VMEM((chunk, dim), jnp.float32),
          pltpu.SemaphoreType.DMA,
      ],
      compiler_params=pltpu.CompilerParams(use_tc_tiling_on_sc=False),
  )
  def gather_kernel(idx_hbm, table_hbm, out_hbm, idx_v, rows_v, sem):
    wid = lax.axis_index("s") * num_cores + lax.axis_index("c")
    base = pl.multiple_of(wid * n_per_w, 8)

    def body(g, carry):
      off = pl.multiple_of(base + g * chunk, 8)
      pltpu.sync_copy(idx_hbm.at[pl.ds(off, chunk)], idx_v)
      pltpu.async_copy(table_hbm.at[idx_v], rows_v, sem).wait()
      pltpu.sync_copy(rows_v, out_hbm.at[pl.ds(off, chunk)])
      return carry

    lax.fori_loop(0, n_chunks, body, 0)

  return gather_kernel


def kernel(indices, embeddings):
  batch, hist = indices.shape
  vocab, dim = embeddings.shape
  n_total = batch * hist
  info = plsc.get_sparse_core_info()
  num_workers = info.num_cores * info.num_subcores
  gather = _make_gather(n_total, vocab, dim, num_workers, info.num_cores,
                        chunk=512)
  out = gather(indices.reshape(n_total), embeddings)
  return out.reshape(batch, hist, dim)


# speedup vs baseline: 1.8605x; 1.0364x over previous
import functools

import jax
import jax.numpy as jnp
from jax import lax
from jax.experimental import pallas as pl
from jax.experimental.pallas import tpu as pltpu
from jax.experimental.pallas import tpu_sc as plsc


def _make_gather(n_total: int, vocab: int, dim: int, num_workers: int,
                 num_cores: int, chunk: int, nbuf: int):
  n_per_w = n_total // num_workers
  n_chunks = n_per_w // chunk
  assert n_per_w % chunk == 0 and n_chunks % nbuf == 0 and chunk % 8 == 0
  n_outer = n_chunks // nbuf
  mesh = plsc.VectorSubcoreMesh(core_axis_name="c", subcore_axis_name="s")

  @functools.partial(
      pl.kernel,
      out_type=jax.ShapeDtypeStruct((n_total, dim), jnp.float32),
      mesh=mesh,
      scratch_types=[
          pltpu.VMEM((nbuf, chunk), jnp.int32),
          pltpu.VMEM((nbuf, chunk, dim), jnp.float32),
      ] + [pltpu.SemaphoreType.DMA] * (2 * nbuf),
      compiler_params=pltpu.CompilerParams(use_tc_tiling_on_sc=False),
  )
  def gather_kernel(idx_hbm, table_hbm, out_hbm, idx_v, rows_v, *sems):
    gsems, wsems = sems[:nbuf], sems[nbuf:]
    wid = lax.axis_index("s") * num_cores + lax.axis_index("c")
    base = pl.multiple_of(wid * n_per_w, 8)

    def off(g):
      return pl.multiple_of(base + g * chunk, 8)

    def start(g, b):
      pltpu.sync_copy(idx_hbm.at[pl.ds(off(g), chunk)], idx_v.at[b])
      pltpu.async_copy(table_hbm.at[idx_v.at[b]], rows_v.at[b], gsems[b])

    def wait_gather(b):
      pltpu.make_async_copy(table_hbm.at[idx_v.at[b]], rows_v.at[b],
                            gsems[b]).wait()

    def start_write(g, b):
      pltpu.async_copy(rows_v.at[b], out_hbm.at[pl.ds(off(g), chunk)],
                       wsems[b])

    def wait_write(g, b):
      pltpu.make_async_copy(rows_v.at[b], out_hbm.at[pl.ds(off(g), chunk)],
                            wsems[b]).wait()

    for b in range(nbuf):
      start(b, b)

    def body(t, carry):
      g0 = t * nbuf
      for b in range(nbuf):
        p = g0 + b
        wait_gather(b)
        start_write(p, b)
        nxt = p + nbuf

        @pl.when(nxt < n_chunks)
        def _refill():
          wait_write(p, b)
          start(nxt, b)
      return carry

    lax.fori_loop(0, n_outer, body, 0)

    for b in range(nbuf):
      wait_write(n_chunks - nbuf + b, b)

  return gather_kernel


def kernel(indices, embeddings):
  batch, hist = indices.shape
  vocab, dim = embeddings.shape
  n_total = batch * hist
  info = plsc.get_sparse_core_info()
  num_workers = info.num_cores * info.num_subcores
  gather = _make_gather(n_total, vocab, dim, num_workers, info.num_cores,
                        chunk=400, nbuf=4)
  out = gather(indices.reshape(n_total), embeddings)
  return out.reshape(batch, hist, dim)


# speedup vs baseline: 1.9567x; 1.0517x over previous
import functools

import jax
import jax.numpy as jnp
from jax import lax
from jax.experimental import pallas as pl
from jax.experimental.pallas import tpu as pltpu
from jax.experimental.pallas import tpu_sc as plsc


def _make_gather(batch: int, hist: int, dim: int, num_workers: int,
                 num_cores: int, kb: int, nbuf: int):
  n_total = batch * hist
  n_per_w = n_total // num_workers
  chunk = kb * hist
  n_chunks = n_per_w // chunk
  assert n_per_w % chunk == 0 and n_chunks % nbuf == 0 and chunk % 8 == 0
  n_outer = n_chunks // nbuf
  b_per_w = batch // num_workers
  mesh = plsc.VectorSubcoreMesh(core_axis_name="c", subcore_axis_name="s")

  @functools.partial(
      pl.kernel,
      out_type=jax.ShapeDtypeStruct((batch, hist, dim), jnp.float32),
      mesh=mesh,
      scratch_types=[
          pltpu.VMEM((nbuf, chunk), jnp.int32),
          pltpu.VMEM((nbuf, chunk, dim), jnp.float32),
      ] + [pltpu.SemaphoreType.DMA] * (2 * nbuf),
      compiler_params=pltpu.CompilerParams(use_tc_tiling_on_sc=False),
  )
  def gather_kernel(idx_hbm, table_hbm, out_hbm, idx_v, rows_v, *sems):
    gsems, wsems = sems[:nbuf], sems[nbuf:]
    wid = lax.axis_index("s") * num_cores + lax.axis_index("c")
    base = pl.multiple_of(wid * n_per_w, 8)
    bbase = wid * b_per_w

    def start(g, b):
      off = pl.multiple_of(base + g * chunk, 8)
      pltpu.sync_copy(idx_hbm.at[pl.ds(off, chunk)], idx_v.at[b])
      pltpu.async_copy(table_hbm.at[idx_v.at[b]], rows_v.at[b], gsems[b])

    def wait_gather(b):
      pltpu.make_async_copy(table_hbm.at[idx_v.at[b]], rows_v.at[b],
                            gsems[b]).wait()

    def start_write(g, b):
      for k in range(kb):
        pltpu.async_copy(rows_v.at[b, pl.ds(k * hist, hist)],
                         out_hbm.at[bbase + g * kb + k], wsems[b])

    def wait_write(g, b):
      for k in range(kb):
        pltpu.make_async_copy(rows_v.at[b, pl.ds(k * hist, hist)],
                              out_hbm.at[bbase + g * kb + k],
                              wsems[b]).wait()

    for b in range(nbuf):
      start(b, b)

    def body(t, carry):
      g0 = t * nbuf
      for b in range(nbuf):
        p = g0 + b
        wait_gather(b)
        start_write(p, b)
        nxt = p + nbuf

        @pl.when(nxt < n_chunks)
        def _refill():
          wait_write(p, b)
          start(nxt, b)
      return carry

    lax.fori_loop(0, n_outer, body, 0)

    for b in range(nbuf):
      wait_write(n_chunks - nbuf + b, b)

  return gather_kernel


def kernel(indices, embeddings):
  batch, hist = indices.shape
  vocab, dim = embeddings.shape
  info = plsc.get_sparse_core_info()
  num_workers = info.num_cores * info.num_subcores
  table2 = jnp.pad(embeddings, ((0, 0), (0, 128 - dim))).reshape(-1, dim)
  idx2 = indices.reshape(batch * hist) * 2
  gather = _make_gather(batch, hist, dim, num_workers, info.num_cores,
                        kb=8, nbuf=4)
  return gather(idx2, table2)
